# baseline (speedup 1.0000x reference)
import functools

import jax
import jax.numpy as jnp
import numpy as np
from jax import lax
from jax.experimental import pallas as pl
from jax.experimental.pallas import tpu as pltpu
from jax.experimental.pallas import tpu_sc as plsc


def _node_tables_body(x_ref, batch_ref, instr_ref, w1s_ref, w1d_ref,
                      w1u_ref, t_ref):
    num_graphs = instr_ref.shape[0]
    c = jnp.dot(instr_ref[...], w1u_ref[...],
                preferred_element_type=jnp.float32)
    onehot = (batch_ref[...][:, None] ==
              lax.broadcasted_iota(jnp.int32, (1, num_graphs), 1)
              ).astype(jnp.float32)
    p = (jnp.dot(x_ref[...], w1s_ref[...],
                 preferred_element_type=jnp.float32)
         + jnp.dot(onehot, c, preferred_element_type=jnp.float32))
    q = jnp.dot(x_ref[...], w1d_ref[...], preferred_element_type=jnp.float32)
    t_ref[...] = jnp.concatenate([p, q], axis=1)


def _edge_r_body(ea_ref, w1e_ref, b1_ref, r_ref):
    r_ref[...] = (jnp.dot(ea_ref[...], w1e_ref[...],
                          preferred_element_type=jnp.float32)
                  + b1_ref[...]).astype(jnp.bfloat16)


_CH = 40
_NBUF = 5


def _make_sc_edge(n_edges, h, nc, ns):
    nw = nc * ns
    epw = n_edges // nw
    nchunk = epw // _CH
    assert nchunk % _NBUF == 0
    mesh = plsc.VectorSubcoreMesh(core_axis_name="c", subcore_axis_name="s",
                                  num_cores=nc, num_subcores=ns)

    ngrp = (_CH + 15) // 16

    buf_types = []
    for _ in range(_NBUF):
        buf_types += [
            pltpu.VMEM((_CH,), jnp.int32),
            pltpu.VMEM((_CH,), jnp.int32),
            pltpu.VMEM((_CH, 2 * h), jnp.float32),
            pltpu.VMEM((_CH, 2 * h), jnp.float32),
            pltpu.VMEM((_CH, h // 2), jnp.int32),
            pltpu.VMEM((ngrp * 16,), jnp.float32),
            pltpu.SemaphoreType.DMA,
            pltpu.SemaphoreType.DMA,
            pltpu.SemaphoreType.DMA,
        ]

    @functools.partial(
        pl.kernel,
        out_type=jax.ShapeDtypeStruct((n_edges,), jnp.float32),
        mesh=mesh,
        scratch_types=[
            pltpu.VMEM((ngrp * 256,), jnp.float32),
            pltpu.VMEM((2 * h,), jnp.float32),
        ] + buf_types,
        compiler_params=pltpu.CompilerParams(needs_layout_passes=False),
    )
    def sc_edge(t_hbm, r_hbm, src_hbm, dst_hbm, wb_hbm, out_hbm,
                mbuf, wbv, *bufs):
        sidx = bufs[0::9]
        didx = bufs[1::9]
        srow = bufs[2::9]
        drow = bufs[3::9]
        rrow = bufs[4::9]
        obuf = bufs[5::9]
        isem = bufs[6::9]
        gsem = bufs[7::9]
        osem = bufs[8::9]

        wid = lax.axis_index("s") * nc + lax.axis_index("c")
        base = wid * epw
        pltpu.sync_copy(wb_hbm, wbv)
        b2vec = wbv[pl.ds(h, 16)]
        lane16 = lax.iota(jnp.int32, 16) * 16

        def idx_issue(ci, b):
            cb = base + ci * _CH
            pltpu.async_copy(src_hbm.at[pl.ds(cb, _CH)], sidx[b], isem[b])
            pltpu.async_copy(dst_hbm.at[pl.ds(cb, _CH)], didx[b], isem[b])

        def idx_wait(ci, b):
            cb = base + ci * _CH
            pltpu.make_async_copy(src_hbm.at[pl.ds(cb, _CH)],
                                  sidx[b], isem[b]).wait()
            pltpu.make_async_copy(dst_hbm.at[pl.ds(cb, _CH)],
                                  didx[b], isem[b]).wait()

        def gather_issue(ci, b):
            cb = base + ci * _CH
            pltpu.async_copy(t_hbm.at[sidx[b]], srow[b], gsem[b])
            pltpu.async_copy(t_hbm.at[didx[b]], drow[b], gsem[b])
            pltpu.async_copy(r_hbm.at[pl.ds(cb, _CH), :], rrow[b], gsem[b])

        def gather_wait(ci, b):
            cb = base + ci * _CH
            pltpu.make_async_copy(t_hbm.at[sidx[b]], srow[b], gsem[b]).wait()
            pltpu.make_async_copy(t_hbm.at[didx[b]], drow[b], gsem[b]).wait()
            pltpu.make_async_copy(r_hbm.at[pl.ds(cb, _CH), :],
                                  rrow[b], gsem[b]).wait()

        def out_wait(ci, b):
            pltpu.make_async_copy(obuf[b].at[pl.ds(0, _CH)],
                                  out_hbm.at[pl.ds(base + ci * _CH, _CH)],
                                  osem[b]).wait()

        def compute(ci, b):
            def edge_body(e, ecarry):
                acc = None
                for j in range(h // 32):
                    ri = rrow[b][e, pl.ds(16 * j, 16)]
                    rb = plsc.bitcast(ri, jnp.bfloat16)
                    rs = plsc.unpack(rb, format=plsc.PackFormat.INTERLEAVED)
                    for k in range(2):
                        jj = 2 * j + k
                        sl = pl.ds(16 * jj, 16)
                        t = (srow[b][e, sl] + drow[b][e, pl.ds(h + 16 * jj, 16)]
                             + rs[k])
                        s = t / (1.0 + jnp.exp(-t))
                        m = s * wbv[sl]
                        acc = m if acc is None else acc + m
                mbuf[pl.ds(e * 16, 16)] = acc
                return ecarry

            lax.fori_loop(0, _CH, edge_body, 0, unroll=2)

            def grp_body(gi, gcarry):
                acc = b2vec
                for j in range(16):
                    flat = gi * 256 + lane16 + j
                    acc = acc + plsc.load_gather(mbuf, [flat])
                obuf[b][pl.ds(gi * 16, 16)] = acc
                return gcarry

            lax.fori_loop(0, ngrp, grp_body, 0)
            pltpu.async_copy(obuf[b].at[pl.ds(0, _CH)],
                             out_hbm.at[pl.ds(base + ci * _CH, _CH)],
                             osem[b])

        for b in range(_NBUF):
            idx_issue(b, b)
        for b in range(_NBUF - 1):
            idx_wait(b, b)
            gather_issue(b, b)

        def outer_body(oi, carry):
            for b in range(_NBUF):
                s = oi * _NBUF + b
                gather_wait(s, b)

                @pl.when(oi > 0)
                def _():
                    out_wait(s - _NBUF, b)

                compute(s, b)

                bg = (b + _NBUF - 1) % _NBUF

                @pl.when(s + _NBUF - 1 < nchunk)
                def _():
                    idx_wait(s + _NBUF - 1, bg)
                    gather_issue(s + _NBUF - 1, bg)

                @pl.when(s + _NBUF < nchunk)
                def _():
                    idx_issue(s + _NBUF, b)
            return carry

        lax.fori_loop(0, nchunk // _NBUF, outer_body, 0)
        for b in range(_NBUF):
            out_wait(nchunk - _NBUF + b, b)

    return sc_edge


def kernel(instr, x, edge_index, batch, edge_attr, W1, b1, W2, b2):
    n, d = x.shape
    g, di = instr.shape
    e, de = edge_attr.shape
    h = W1.shape[1]

    src = edge_index[0].astype(jnp.int32)
    dst = edge_index[1].astype(jnp.int32)
    batch32 = batch.astype(jnp.int32)

    pt = np.concatenate(
        [np.stack([np.arange(g, g + 16), np.arange(g + 16, g + 32)], axis=1)
         .reshape(-1) for g in range(0, h, 32)])

    w1s = W1[:d]
    w1d = W1[d:2 * d]
    w1u = W1[2 * d:2 * d + di]
    w1e = W1[2 * d + di:][:, pt]
    b1p = b1[pt]

    t_tab = pl.pallas_call(
        _node_tables_body,
        out_shape=jax.ShapeDtypeStruct((n, 2 * h), jnp.float32),
    )(x, batch32, instr, w1s, w1d, w1u)

    blk = 8000
    r_tab = pl.pallas_call(
        _edge_r_body,
        grid=(e // blk,),
        in_specs=[pl.BlockSpec((blk, de), lambda i: (i, 0)),
                  pl.BlockSpec((de, h), lambda i: (0, 0)),
                  pl.BlockSpec((1, h), lambda i: (0, 0))],
        out_specs=pl.BlockSpec((blk, h), lambda i: (i, 0)),
        out_shape=jax.ShapeDtypeStruct((e, h), jnp.bfloat16),
    )(edge_attr, w1e, b1p[None, :])

    wb = jnp.concatenate([W2[:, 0], jnp.broadcast_to(b2, (h,))])

    r32 = lax.bitcast_convert_type(r_tab.reshape(e, h // 2, 2), jnp.int32)

    sc_edge = _make_sc_edge(e, h, 2, 16)
    return sc_edge(t_tab, r32, src, dst, wb)


# speedup vs baseline: 1.9005x; 1.9005x over previous
import functools

import jax
import jax.numpy as jnp
import numpy as np
from jax import lax
from jax.experimental import pallas as pl
from jax.experimental.pallas import tpu as pltpu
from jax.experimental.pallas import tpu_sc as plsc


def _node_tables_body(x_ref, batch_ref, instr_ref, w1s_ref, w1d_ref,
                      w1u_ref, t_ref):
    num_graphs = instr_ref.shape[0]
    c = jnp.dot(instr_ref[...], w1u_ref[...],
                preferred_element_type=jnp.float32)
    onehot = (batch_ref[...][:, None] ==
              lax.broadcasted_iota(jnp.int32, (1, num_graphs), 1)
              ).astype(jnp.float32)
    p = (jnp.dot(x_ref[...], w1s_ref[...],
                 preferred_element_type=jnp.float32)
         + jnp.dot(onehot, c, preferred_element_type=jnp.float32))
    q = jnp.dot(x_ref[...], w1d_ref[...], preferred_element_type=jnp.float32)
    t_ref[...] = jnp.concatenate([p, q], axis=1)


def _edge_r_body(ea_ref, w1e_ref, b1_ref, r_ref):
    r_ref[...] = (jnp.dot(ea_ref[...], w1e_ref[...],
                          preferred_element_type=jnp.float32)
                  + b1_ref[...])


_CH = 80
_NBUF = 4


def _make_sc_edge(n_edges, h, nc, ns):
    nw = nc * ns
    epw = n_edges // nw
    nchunk = epw // _CH
    mesh = plsc.VectorSubcoreMesh(core_axis_name="c", subcore_axis_name="s",
                                  num_cores=nc, num_subcores=ns)

    ngrp = (_CH + 15) // 16

    buf_types = []
    for _ in range(_NBUF):
        buf_types += [
            pltpu.VMEM((_CH,), jnp.int32),
            pltpu.VMEM((_CH,), jnp.int32),
            pltpu.VMEM((_CH, 2 * h), jnp.float32),
            pltpu.VMEM((_CH, 2 * h), jnp.float32),
            pltpu.VMEM((_CH, h), jnp.float32),
            pltpu.VMEM((ngrp * 16,), jnp.float32),
            pltpu.SemaphoreType.DMA,
            pltpu.SemaphoreType.DMA,
            pltpu.SemaphoreType.DMA,
        ]

    @functools.partial(
        pl.kernel,
        out_type=jax.ShapeDtypeStruct((n_edges,), jnp.float32),
        mesh=mesh,
        scratch_types=[
            pltpu.VMEM((ngrp * 256,), jnp.float32),
            pltpu.VMEM((2 * h,), jnp.float32),
        ] + buf_types,
        compiler_params=pltpu.CompilerParams(needs_layout_passes=False),
    )
    def sc_edge(t_hbm, r_hbm, src_hbm, dst_hbm, wb_hbm, out_hbm,
                mbuf, wbv, *bufs):
        sidx = bufs[0::9]
        didx = bufs[1::9]
        srow = bufs[2::9]
        drow = bufs[3::9]
        rrow = bufs[4::9]
        obuf = bufs[5::9]
        isem = bufs[6::9]
        gsem = bufs[7::9]
        osem = bufs[8::9]

        wid = lax.axis_index("s") * nc + lax.axis_index("c")
        base = wid * epw
        pltpu.sync_copy(wb_hbm, wbv)
        b2vec = wbv[pl.ds(h, 16)]
        lane16 = lax.iota(jnp.int32, 16) * 16

        def idx_issue(ci, b):
            cb = base + ci * _CH
            pltpu.async_copy(src_hbm.at[pl.ds(cb, _CH)], sidx[b], isem[b])
            pltpu.async_copy(dst_hbm.at[pl.ds(cb, _CH)], didx[b], isem[b])

        def idx_wait(ci, b):
            cb = base + ci * _CH
            pltpu.make_async_copy(src_hbm.at[pl.ds(cb, _CH)],
                                  sidx[b], isem[b]).wait()
            pltpu.make_async_copy(dst_hbm.at[pl.ds(cb, _CH)],
                                  didx[b], isem[b]).wait()

        def gather_issue(ci, b):
            cb = base + ci * _CH
            pltpu.async_copy(t_hbm.at[sidx[b]], srow[b], gsem[b])
            pltpu.async_copy(t_hbm.at[didx[b]], drow[b], gsem[b])
            pltpu.async_copy(r_hbm.at[pl.ds(cb, _CH), :], rrow[b], gsem[b])

        def gather_wait(ci, b):
            cb = base + ci * _CH
            pltpu.make_async_copy(t_hbm.at[sidx[b]], srow[b], gsem[b]).wait()
            pltpu.make_async_copy(t_hbm.at[didx[b]], drow[b], gsem[b]).wait()
            pltpu.make_async_copy(r_hbm.at[pl.ds(cb, _CH), :],
                                  rrow[b], gsem[b]).wait()

        def out_wait(ci, b):
            pltpu.make_async_copy(obuf[b].at[pl.ds(0, _CH)],
                                  out_hbm.at[pl.ds(base + ci * _CH, _CH)],
                                  osem[b]).wait()

        def compute(ci, b):
            def edge_body(e, ecarry):
                acc = None
                for j in range(h // 16):
                    sl = pl.ds(16 * j, 16)
                    t = (srow[b][e, sl] + drow[b][e, pl.ds(h + 16 * j, 16)]
                         + rrow[b][e, sl])
                    s = t / (1.0 + jnp.exp(-t))
                    m = s * wbv[sl]
                    acc = m if acc is None else acc + m
                mbuf[pl.ds(e * 16, 16)] = acc
                return ecarry

            lax.fori_loop(0, _CH, edge_body, 0, unroll=2)

            def grp_body(gi, gcarry):
                acc = b2vec
                for j in range(16):
                    flat = gi * 256 + lane16 + j
                    acc = acc + plsc.load_gather(mbuf, [flat])
                obuf[b][pl.ds(gi * 16, 16)] = acc
                return gcarry

            lax.fori_loop(0, ngrp, grp_body, 0)
            pltpu.async_copy(obuf[b].at[pl.ds(0, _CH)],
                             out_hbm.at[pl.ds(base + ci * _CH, _CH)],
                             osem[b])

        for b in range(_NBUF):
            idx_issue(b, b)
        for b in range(_NBUF - 1):
            idx_wait(b, b)
            gather_issue(b, b)

        nfull = nchunk // _NBUF

        def outer_body(oi, carry):
            for b in range(_NBUF):
                s = oi * _NBUF + b
                gather_wait(s, b)

                @pl.when(oi > 0)
                def _():
                    out_wait(s - _NBUF, b)

                compute(s, b)

                bg = (b + _NBUF - 1) % _NBUF

                @pl.when(s + _NBUF - 1 < nchunk)
                def _():
                    idx_wait(s + _NBUF - 1, bg)
                    gather_issue(s + _NBUF - 1, bg)

                @pl.when(s + _NBUF < nchunk)
                def _():
                    idx_issue(s + _NBUF, b)
            return carry

        lax.fori_loop(0, nfull, outer_body, 0)

        for s in range(nfull * _NBUF, nchunk):
            b = s % _NBUF
            gather_wait(s, b)
            if s >= _NBUF:
                out_wait(s - _NBUF, b)
            compute(s, b)
            bg = (b + _NBUF - 1) % _NBUF
            if s + _NBUF - 1 < nchunk:
                idx_wait(s + _NBUF - 1, bg)
                gather_issue(s + _NBUF - 1, bg)
            if s + _NBUF < nchunk:
                idx_issue(s + _NBUF, b)

        for ci in range(max(nchunk - _NBUF, 0), nchunk):
            out_wait(ci, ci % _NBUF)

    return sc_edge


def kernel(instr, x, edge_index, batch, edge_attr, W1, b1, W2, b2):
    n, d = x.shape
    g, di = instr.shape
    e, de = edge_attr.shape
    h = W1.shape[1]

    src = edge_index[0].astype(jnp.int32)
    dst = edge_index[1].astype(jnp.int32)
    batch32 = batch.astype(jnp.int32)

    w1s = W1[:d]
    w1d = W1[d:2 * d]
    w1u = W1[2 * d:2 * d + di]
    w1e = W1[2 * d + di:]

    t_tab = pl.pallas_call(
        _node_tables_body,
        out_shape=jax.ShapeDtypeStruct((n, 2 * h), jnp.float32),
    )(x, batch32, instr, w1s, w1d, w1u)

    blk = 8000
    r_tab = pl.pallas_call(
        _edge_r_body,
        grid=(e // blk,),
        in_specs=[pl.BlockSpec((blk, de), lambda i: (i, 0)),
                  pl.BlockSpec((de, h), lambda i: (0, 0)),
                  pl.BlockSpec((1, h), lambda i: (0, 0))],
        out_specs=pl.BlockSpec((blk, h), lambda i: (i, 0)),
        out_shape=jax.ShapeDtypeStruct((e, h), jnp.float32),
    )(edge_attr, w1e, b1[None, :])

    wb = jnp.concatenate([W2[:, 0], jnp.broadcast_to(b2, (h,))])

    sc_edge = _make_sc_edge(e, h, 2, 16)
    return sc_edge(t_tab, r_tab, src, dst, wb)
